# Initial kernel scaffold; baseline (speedup 1.0000x reference)
import jax
import jax.numpy as jnp
from jax.experimental import pallas as pl


def kernel(D, W, Wp, ctx_ids, doc_ids, target_and_noise_ids):
    raise NotImplementedError("write your pallas kernel here")


# speedup vs baseline: 2.5688x; 2.5688x over previous
import jax
import jax.numpy as jnp
from jax import lax
from jax.experimental import pallas as pl
from jax.experimental.pallas import tpu as pltpu

_GRP = 8
_TB = 256


def _dm_gather_kernel(ids_ref, x0_ref, w_ref, wp_ref, out_ref, ws_ref, g_ref):
    tb, s_dim = out_ref.shape
    n_ids = ids_ref.shape[1]
    c_dim = n_ids - s_dim

    def group(g, carry):
        base = pl.multiple_of(g * _GRP, _GRP)
        for j in range(_GRP):
            b = base + j
            acc = w_ref[ids_ref[b, 0], 0]
            for c in range(1, c_dim):
                acc = acc + w_ref[ids_ref[b, c], 0]
            ws_ref[j] = acc
        for s in range(s_dim):
            for j in range(_GRP):
                b = base + j
                g_ref[s * _GRP + j] = wp_ref[ids_ref[b, c_dim + s], 0]
        x8 = x0_ref[pl.ds(base, _GRP), :] + ws_ref[...]
        cols = []
        for s in range(s_dim):
            gs = g_ref[s * _GRP:(s + 1) * _GRP, :]
            cols.append(jnp.sum(x8 * gs, axis=1, keepdims=True))
        out_ref[pl.ds(base, _GRP), :] = jnp.concatenate(cols, axis=1)
        return carry

    lax.fori_loop(0, tb // _GRP, group, 0)


def kernel(D, W, Wp, ctx_ids, doc_ids, target_and_noise_ids):
    B, C = ctx_ids.shape
    S = target_and_noise_ids.shape[1]
    E, Nw = Wp.shape
    Nwp1 = W.shape[0]

    x0 = D.astype(jnp.float32)[doc_ids]
    ids = jnp.concatenate(
        [ctx_ids.astype(jnp.int32),
         target_and_noise_ids.astype(jnp.int32)], axis=1)

    w3 = W.astype(jnp.float32).reshape(Nwp1, 1, E)
    wpt = Wp.astype(jnp.float32).T.reshape(Nw, 1, E)

    n_blocks = pl.cdiv(B, _TB)
    b_pad = n_blocks * _TB - B
    if b_pad:
        x0 = jnp.pad(x0, ((0, b_pad), (0, 0)))
        ids = jnp.pad(ids, ((0, b_pad), (0, 0)))

    out = pl.pallas_call(
        _dm_gather_kernel,
        out_shape=jax.ShapeDtypeStruct((n_blocks * _TB, S), jnp.float32),
        grid=(n_blocks,),
        in_specs=[
            pl.BlockSpec((_TB, C + S), lambda b: (b, 0),
                         memory_space=pltpu.SMEM),
            pl.BlockSpec((_TB, E), lambda b: (b, 0)),
            pl.BlockSpec((Nwp1, 1, E), lambda b: (0, 0, 0)),
            pl.BlockSpec((Nw, 1, E), lambda b: (0, 0, 0)),
        ],
        out_specs=pl.BlockSpec((_TB, S), lambda b: (b, 0)),
        scratch_shapes=[
            pltpu.VMEM((_GRP, E), jnp.float32),
            pltpu.VMEM((S * _GRP, E), jnp.float32),
        ],
        compiler_params=pltpu.CompilerParams(
            dimension_semantics=("parallel",),
            vmem_limit_bytes=46 * 1024 * 1024,
        ),
    )(ids, x0, w3, wpt)

    return out[:B]


# speedup vs baseline: 2.5745x; 1.0022x over previous
import jax
import jax.numpy as jnp
from jax import lax
from jax.experimental import pallas as pl
from jax.experimental.pallas import tpu as pltpu

_GRP = 16
_TB = 256


def _dm_gather_kernel(ids_ref, x0_ref, w_ref, wp_ref, out_ref, ws_ref, g_ref):
    tb, s_dim = out_ref.shape
    n_ids = ids_ref.shape[1]
    c_dim = n_ids - s_dim

    def group(g, carry):
        base = pl.multiple_of(g * _GRP, _GRP)
        row_ids = [[ids_ref[base + j, c] for c in range(n_ids)]
                   for j in range(_GRP)]
        for j in range(_GRP):
            acc = w_ref[row_ids[j][0], 0]
            for c in range(1, c_dim):
                acc = acc + w_ref[row_ids[j][c], 0]
            ws_ref[j] = acc
        for s in range(s_dim):
            for j in range(_GRP):
                g_ref[s * _GRP + j] = wp_ref[row_ids[j][c_dim + s], 0]
        xg = x0_ref[pl.ds(base, _GRP), :] + ws_ref[...]
        cols = []
        for s in range(s_dim):
            gs = g_ref[s * _GRP:(s + 1) * _GRP, :]
            cols.append(jnp.sum(xg * gs, axis=1, keepdims=True))
        out_ref[pl.ds(base, _GRP), :] = jnp.concatenate(cols, axis=1)
        return carry

    lax.fori_loop(0, tb // _GRP, group, 0)


def kernel(D, W, Wp, ctx_ids, doc_ids, target_and_noise_ids):
    B, C = ctx_ids.shape
    S = target_and_noise_ids.shape[1]
    E, Nw = Wp.shape
    Nwp1 = W.shape[0]

    x0 = D.astype(jnp.float32)[doc_ids]
    ids = jnp.concatenate(
        [ctx_ids.astype(jnp.int32),
         target_and_noise_ids.astype(jnp.int32)], axis=1)

    w3 = W.astype(jnp.float32).reshape(Nwp1, 1, E)
    wpt = Wp.astype(jnp.float32).T.reshape(Nw, 1, E)

    n_blocks = pl.cdiv(B, _TB)
    b_pad = n_blocks * _TB - B
    if b_pad:
        x0 = jnp.pad(x0, ((0, b_pad), (0, 0)))
        ids = jnp.pad(ids, ((0, b_pad), (0, 0)))

    out = pl.pallas_call(
        _dm_gather_kernel,
        out_shape=jax.ShapeDtypeStruct((n_blocks * _TB, S), jnp.float32),
        grid=(n_blocks,),
        in_specs=[
            pl.BlockSpec((_TB, C + S), lambda b: (b, 0),
                         memory_space=pltpu.SMEM),
            pl.BlockSpec((_TB, E), lambda b: (b, 0)),
            pl.BlockSpec((Nwp1, 1, E), lambda b: (0, 0, 0)),
            pl.BlockSpec((Nw, 1, E), lambda b: (0, 0, 0)),
        ],
        out_specs=pl.BlockSpec((_TB, S), lambda b: (b, 0)),
        scratch_shapes=[
            pltpu.VMEM((_GRP, E), jnp.float32),
            pltpu.VMEM((S * _GRP, E), jnp.float32),
        ],
        compiler_params=pltpu.CompilerParams(
            dimension_semantics=("arbitrary",),
            vmem_limit_bytes=46 * 1024 * 1024,
        ),
    )(ids, x0, w3, wpt)

    return out[:B]


# speedup vs baseline: 2.9381x; 1.1412x over previous
import jax
import jax.numpy as jnp
from jax import lax
from jax.experimental import pallas as pl
from jax.experimental.pallas import tpu as pltpu

_GRP = 16
_TB = 256


def _dm_gather_kernel(ids_ref, x0_ref, w_ref, wp_ref, out_ref, ws_ref, g_ref):
    tb, s_dim = out_ref.shape
    n_ids = ids_ref.shape[1]
    c_dim = n_ids - s_dim

    def group(g, carry):
        base = pl.multiple_of(g * _GRP, _GRP)
        for j in range(_GRP):
            b = base + j
            acc = w_ref[ids_ref[b, 0], 0]
            for c in range(1, c_dim):
                acc = acc + w_ref[ids_ref[b, c], 0]
            ws_ref[j] = acc
        for s in range(s_dim):
            for j in range(_GRP):
                b = base + j
                g_ref[s * _GRP + j] = wp_ref[ids_ref[b, c_dim + s], 0]
        xg = x0_ref[pl.ds(base, _GRP), :] + ws_ref[...]
        cols = []
        for s in range(s_dim):
            gs = g_ref[s * _GRP:(s + 1) * _GRP, :]
            cols.append(jnp.sum(xg * gs, axis=1, keepdims=True))
        out_ref[pl.ds(base, _GRP), :] = jnp.concatenate(cols, axis=1)
        return carry

    lax.fori_loop(0, tb // _GRP, group, 0)


def kernel(D, W, Wp, ctx_ids, doc_ids, target_and_noise_ids):
    B, C = ctx_ids.shape
    S = target_and_noise_ids.shape[1]
    E, Nw = Wp.shape
    Nwp1 = W.shape[0]

    x0 = D.astype(jnp.float32)[doc_ids]
    ids = jnp.concatenate(
        [ctx_ids.astype(jnp.int32),
         target_and_noise_ids.astype(jnp.int32)], axis=1)

    w3 = W.astype(jnp.float32).reshape(Nwp1, 1, E)
    wpt = Wp.astype(jnp.float32).T.reshape(Nw, 1, E)

    n_blocks = pl.cdiv(B, _TB)
    b_pad = n_blocks * _TB - B
    if b_pad:
        x0 = jnp.pad(x0, ((0, b_pad), (0, 0)))
        ids = jnp.pad(ids, ((0, b_pad), (0, 0)))

    out = pl.pallas_call(
        _dm_gather_kernel,
        out_shape=jax.ShapeDtypeStruct((n_blocks * _TB, S), jnp.float32),
        grid=(n_blocks,),
        in_specs=[
            pl.BlockSpec((_TB, C + S), lambda b: (b, 0),
                         memory_space=pltpu.SMEM),
            pl.BlockSpec((_TB, E), lambda b: (b, 0)),
            pl.BlockSpec((Nwp1, 1, E), lambda b: (0, 0, 0)),
            pl.BlockSpec((Nw, 1, E), lambda b: (0, 0, 0)),
        ],
        out_specs=pl.BlockSpec((_TB, S), lambda b: (b, 0)),
        scratch_shapes=[
            pltpu.VMEM((_GRP, E), jnp.float32),
            pltpu.VMEM((S * _GRP, E), jnp.float32),
        ],
        compiler_params=pltpu.CompilerParams(
            dimension_semantics=("arbitrary",),
            vmem_limit_bytes=46 * 1024 * 1024,
        ),
    )(ids, x0, w3, wpt)

    return out[:B]
